# baseline (speedup 1.0000x reference)
import functools
import math

import jax
import jax.numpy as jnp
import numpy as np
from jax import lax
from jax.experimental import pallas as pl
from jax.experimental.pallas import tpu as pltpu
from jax.experimental.pallas import tpu_sc as plsc

NC = 2
NS = 16
NW = NC * NS
LANE = 16
CB = 80

_ILV = plsc.PackFormat.INTERLEAVED


def _store_perm(c):
    p = np.zeros((c,), np.int32)
    for j in range(c // 32):
        for t in range(16):
            p[32 * j + 2 * t] = 32 * j + t
            p[32 * j + 2 * t + 1] = 32 * j + 16 + t
    return p


def _lane_perm(c):
    p = np.zeros((c,), np.int32)
    for j in range(c // 32):
        for t in range(16):
            p[32 * j + t] = 32 * j + 2 * t
            p[32 * j + 16 + t] = 32 * j + 2 * t + 1
    return p


def _qkvs_body(x_ref, w_ref, b_ref, q_ref, k_ref, v_ref, s_ref, *, c):
    y = (jnp.dot(x_ref[...], w_ref[...], preferred_element_type=jnp.float32)
         + b_ref[...])
    q_ref[...] = y[:, :c]
    k_ref[...] = y[:, c:2 * c].astype(jnp.bfloat16)
    v_ref[...] = y[:, 2 * c:3 * c].astype(jnp.bfloat16)
    s_ref[...] = y[:, 3 * c:]


def _epi_qkvs_body(op_ref, den_ref, skip_ref, w_ref, b_ref,
                   q_ref, k_ref, v_ref, s_ref, *, c, relu):
    den = den_ref[0] + den_ref[1]
    denw = jnp.where(den == 0.0, 1.0, den)
    z = (op_ref[0] + op_ref[1]) / denw + skip_ref[...]
    if relu:
        z = jnp.maximum(z, 0.0)
    y = jnp.dot(z, w_ref[...], preferred_element_type=jnp.float32) + b_ref[...]
    q_ref[...] = y[:, :c]
    k_ref[...] = y[:, c:2 * c].astype(jnp.bfloat16)
    v_ref[...] = y[:, 2 * c:3 * c].astype(jnp.bfloat16)
    s_ref[...] = y[:, 3 * c:]


def _epi_pq_body(op_ref, den_ref, skip_ref, w_ref, b_ref, p_ref, q_ref, *, c):
    den = den_ref[0] + den_ref[1]
    denw = jnp.where(den == 0.0, 1.0, den)
    z = (op_ref[0] + op_ref[1]) / denw + skip_ref[...]
    y = jnp.dot(z, w_ref[...], preferred_element_type=jnp.float32) + b_ref[...]
    p_ref[...] = y[:, :c].astype(jnp.bfloat16)
    q_ref[...] = y[:, c:].astype(jnp.bfloat16)


def _tc_qkvs(x, w, b, br=1000):
    n, d = x.shape
    c = d
    grid = (n // br,)
    bf = functools.partial(_qkvs_body, c=c)
    outs = [jax.ShapeDtypeStruct((n, c), jnp.float32),
            jax.ShapeDtypeStruct((n, c), jnp.bfloat16),
            jax.ShapeDtypeStruct((n, c), jnp.bfloat16),
            jax.ShapeDtypeStruct((n, c), jnp.float32)]
    return pl.pallas_call(
        bf,
        grid=grid,
        in_specs=[
            pl.BlockSpec((br, d), lambda i: (i, 0)),
            pl.BlockSpec((d, 4 * c), lambda i: (0, 0)),
            pl.BlockSpec((1, 4 * c), lambda i: (0, 0)),
        ],
        out_specs=[pl.BlockSpec((br, c), lambda i: (i, 0))] * 4,
        out_shape=outs,
    )(x, w, b.reshape(1, 4 * c))


def _tc_epi_qkvs(op, den, skip, w, b, relu, br=1000):
    n, c = skip.shape
    grid = (n // br,)
    bf = functools.partial(_epi_qkvs_body, c=c, relu=relu)
    outs = [jax.ShapeDtypeStruct((n, c), jnp.float32),
            jax.ShapeDtypeStruct((n, c), jnp.bfloat16),
            jax.ShapeDtypeStruct((n, c), jnp.bfloat16),
            jax.ShapeDtypeStruct((n, c), jnp.float32)]
    return pl.pallas_call(
        bf,
        grid=grid,
        in_specs=[
            pl.BlockSpec((2, br, c), lambda i: (0, i, 0)),
            pl.BlockSpec((2, br, 1), lambda i: (0, i, 0)),
            pl.BlockSpec((br, c), lambda i: (i, 0)),
            pl.BlockSpec((c, 4 * c), lambda i: (0, 0)),
            pl.BlockSpec((1, 4 * c), lambda i: (0, 0)),
        ],
        out_specs=[pl.BlockSpec((br, c), lambda i: (i, 0))] * 4,
        out_shape=outs,
    )(op, den.reshape(2, n, 1), skip, w, b.reshape(1, 4 * c))


def _tc_epi_pq(op, den, skip, w, b, br=1000):
    n, c = skip.shape
    grid = (n // br,)
    bf = functools.partial(_epi_pq_body, c=c)
    outs = [jax.ShapeDtypeStruct((n, c), jnp.bfloat16)] * 2
    return pl.pallas_call(
        bf,
        grid=grid,
        in_specs=[
            pl.BlockSpec((2, br, c), lambda i: (0, i, 0)),
            pl.BlockSpec((2, br, 1), lambda i: (0, i, 0)),
            pl.BlockSpec((br, c), lambda i: (i, 0)),
            pl.BlockSpec((c, 2 * c), lambda i: (0, 0)),
            pl.BlockSpec((1, 2 * c), lambda i: (0, 0)),
        ],
        out_specs=[pl.BlockSpec((br, c), lambda i: (i, 0))] * 2,
        out_shape=outs,
    )(op, den.reshape(2, n, 1), skip, w, b.reshape(1, 2 * c))


def _conv_sc_body(q_hbm, kv_hbm, de_hbm, se_hbm, z2_hbm, z1_hbm,
                  outp_hbm, den_hbm,
                  deb, seb, qb, kvb, vsc, exb, out_acc, den_acc,
                  gsem, ssem,
                  *, n_nodes, n_edges, c_dim):
    cid = lax.axis_index("c")
    sid = lax.axis_index("s")
    wid = sid * NC + cid

    @pl.when(sid == 0)
    def _init():
        pltpu.sync_copy(z2_hbm, out_acc)
        pltpu.sync_copy(z1_hbm, den_acc)

    plsc.subcore_barrier()

    rows_pw = n_edges // (NW * CB)

    lane = lax.iota(jnp.int32, LANE)
    inv = 1.0 / math.sqrt(c_dim)
    ngrp = CB // LANE
    nj2 = c_dim // 32

    def chunk_body(c, _):
        ci = pltpu.async_copy(de_hbm.at[wid].at[c], deb.at[0], gsem)
        cj = pltpu.async_copy(se_hbm.at[wid].at[c], seb.at[0], gsem)
        ci.wait()
        cj.wait()
        cq = pltpu.async_copy(q_hbm.at[deb.at[0]], qb, gsem)
        ckv = pltpu.async_copy(kv_hbm.at[seb.at[0]], kvb, gsem)
        cq.wait()
        ckv.wait()

        def grp_body(g, _):
            ex16 = jnp.zeros((LANE,), jnp.float32)
            for e in range(LANE):
                ei = g * LANE + e
                acc = None
                for j in range(nj2):
                    qa = qb[ei, pl.ds(j * 32, LANE)]
                    qb_ = qb[ei, pl.ds(j * 32 + LANE, LANE)]
                    kw = plsc.bitcast(kvb[ei, pl.ds(j * LANE, LANE)],
                                      jnp.bfloat16)
                    ka, kb_ = plsc.unpack(kw, format=_ILV)
                    t = qa * ka + qb_ * kb_
                    acc = t if acc is None else acc + t
                s = jnp.sum(acc) * inv
                exv = jnp.exp(jnp.full((LANE,), s, jnp.float32))
                ex16 = jnp.where(lane == e, exv, ex16)
                for j in range(nj2):
                    vw = plsc.bitcast(
                        kvb[ei, pl.ds(c_dim // 2 + j * LANE, LANE)],
                        jnp.bfloat16)
                    va, vb_ = plsc.unpack(vw, format=_ILV)
                    vsc[ei, pl.ds(j * 32, LANE)] = va * exv
                    vsc[ei, pl.ds(j * 32 + LANE, LANE)] = vb_ * exv
            exb[pl.ds(g * LANE, LANE)] = ex16
            return 0

        lax.fori_loop(0, ngrp, grp_body, 0)

        s1 = pltpu.async_copy(vsc, out_acc.at[deb.at[0]], ssem, add=True)
        s2 = pltpu.async_copy(exb, den_acc.at[deb.at[0]], ssem, add=True)
        s1.wait()
        s2.wait()
        return 0

    lax.fori_loop(0, rows_pw, chunk_body, 0)

    plsc.subcore_barrier()
    rpw = (n_nodes // NS) // 8 * 8
    tail = n_nodes - NS * rpw
    pltpu.sync_copy(out_acc.at[pl.ds(sid * rpw, rpw)],
                    outp_hbm.at[cid].at[pl.ds(sid * rpw, rpw)])

    @pl.when(sid == 0)
    def _den_out():
        if tail:
            pltpu.sync_copy(out_acc.at[pl.ds(NS * rpw, tail)],
                            outp_hbm.at[cid].at[pl.ds(NS * rpw, tail)])
        pltpu.sync_copy(den_acc, den_hbm.at[cid])


def _sc_conv(q, kv, de2, se2, z2, z1):
    n_nodes, c_dim = q.shape
    n_edges = de2.shape[0] * de2.shape[1] * de2.shape[2]
    mesh = plsc.VectorSubcoreMesh(core_axis_name="c", subcore_axis_name="s",
                                  num_cores=NC, num_subcores=NS)
    kern = pl.kernel(
        functools.partial(_conv_sc_body, n_nodes=n_nodes, n_edges=n_edges,
                          c_dim=c_dim),
        compiler_params=pltpu.CompilerParams(needs_layout_passes=False),
        out_type=(
            jax.ShapeDtypeStruct((NC, n_nodes, c_dim), jnp.float32),
            jax.ShapeDtypeStruct((NC, n_nodes), jnp.float32),
        ),
        mesh=mesh,
        scratch_types=[
            pltpu.VMEM((1, CB), jnp.int32),
            pltpu.VMEM((1, CB), jnp.int32),
            pltpu.VMEM((CB, c_dim), jnp.float32),
            pltpu.VMEM((CB, c_dim), jnp.float32),
            pltpu.VMEM((CB, c_dim), jnp.float32),
            pltpu.VMEM((CB,), jnp.float32),
            pltpu.VMEM_SHARED((n_nodes, c_dim), jnp.float32),
            pltpu.VMEM_SHARED((n_nodes,), jnp.float32),
            pltpu.SemaphoreType.DMA,
            pltpu.SemaphoreType.DMA,
        ],
    )
    return kern(q, kv, de2, se2, z2, z1)


def _link_sc_body(pq_hbm, src_hbm, dst_hbm, wb_hbm,
                  out_hbm,
                  srcb, dstb, pb, qb, ob, wbv, gsem,
                  *, c_dim, lp_pw):
    cid = lax.axis_index("c")
    sid = lax.axis_index("s")
    wid = sid * NC + cid

    rows_pw = lp_pw // CB
    row0 = wid * rows_pw
    pltpu.sync_copy(src_hbm.at[wid], srcb)
    pltpu.sync_copy(dst_hbm.at[wid], dstb)
    pltpu.sync_copy(wb_hbm, wbv)

    lane = lax.iota(jnp.int32, LANE)
    nj2 = c_dim // 32
    w2a = [wbv[pl.ds(j * 32, LANE)] for j in range(nj2)]
    w2b = [wbv[pl.ds(j * 32 + LANE, LANE)] for j in range(nj2)]
    b2v = wbv[pl.ds(c_dim, LANE)]
    ngrp = CB // LANE

    def chunk_body(c, _):
        cp = pltpu.async_copy(pq_hbm.at[srcb.at[c]], pb, gsem)
        cq = pltpu.async_copy(pq_hbm.at[dstb.at[c]], qb, gsem)
        cp.wait()
        cq.wait()

        def grp_body(g, _):
            o16 = jnp.zeros((LANE,), jnp.float32)
            for e in range(LANE):
                ei = g * LANE + e
                acc = None
                for j in range(nj2):
                    pw = plsc.bitcast(pb[ei, pl.ds(j * LANE, LANE)],
                                      jnp.bfloat16)
                    qw = plsc.bitcast(
                        qb[ei, pl.ds(c_dim // 2 + j * LANE, LANE)],
                        jnp.bfloat16)
                    pa, pb_ = plsc.unpack(pw, format=_ILV)
                    qa, qb_ = plsc.unpack(qw, format=_ILV)
                    ua = jnp.maximum(pa + qa, 0.0) * w2a[j]
                    ub = jnp.maximum(pb_ + qb_, 0.0) * w2b[j]
                    t = ua + ub
                    acc = t if acc is None else acc + t
                tv = jnp.full((LANE,), jnp.sum(acc), jnp.float32) + b2v
                sg = 1.0 / (1.0 + jnp.exp(-tv))
                o16 = jnp.where(lane == e, sg, o16)
            ob[pl.ds(g * LANE, LANE)] = o16
            return 0

        lax.fori_loop(0, ngrp, grp_body, 0)
        pltpu.sync_copy(ob, out_hbm.at[pl.ds(row0 * CB + c * CB, CB)])
        return 0

    lax.fori_loop(0, rows_pw, chunk_body, 0)


def _sc_link(pq, src2, dst2, wb):
    n_nodes, c_dim = pq.shape
    lp = src2.shape[0] * src2.shape[1] * src2.shape[2]
    lp_pw = lp // NW
    mesh = plsc.VectorSubcoreMesh(core_axis_name="c", subcore_axis_name="s",
                                  num_cores=NC, num_subcores=NS)
    kern = pl.kernel(
        functools.partial(_link_sc_body, c_dim=c_dim, lp_pw=lp_pw),
        compiler_params=pltpu.CompilerParams(needs_layout_passes=False),
        out_type=jax.ShapeDtypeStruct((lp,), jnp.float32),
        mesh=mesh,
        scratch_types=[
            pltpu.VMEM((lp_pw // CB, CB), jnp.int32),
            pltpu.VMEM((lp_pw // CB, CB), jnp.int32),
            pltpu.VMEM((CB, c_dim), jnp.float32),
            pltpu.VMEM((CB, c_dim), jnp.float32),
            pltpu.VMEM((CB,), jnp.float32),
            pltpu.VMEM((c_dim + LANE,), jnp.float32),
            pltpu.SemaphoreType.DMA,
        ],
    )
    return kern(pq, src2, dst2, wb)


def kernel(x, edge_index, src, dst, params):
    n, c = x.shape
    e = edge_index.shape[1]
    l = src.shape[0]

    de2 = edge_index[1].reshape(NW, e // (NW * CB), CB)
    se2 = edge_index[0].reshape(NW, e // (NW * CB), CB)
    z2 = jnp.zeros((n, c), jnp.float32)
    z1 = jnp.zeros((n,), jnp.float32)

    layers = params["layers"]
    lp = params["lp"]
    sperm = _store_perm(c)
    lperm = _lane_perm(c)

    def wcat(p):
        w = jnp.concatenate(
            [p["Wq"][:, lperm], p["Wk"], p["Wv"][:, sperm], p["Wskip"]],
            axis=1)
        b = jnp.concatenate(
            [p["bq"][lperm], p["bk"], p["bv"][sperm], p["bskip"]])
        return w, b

    def pk(a):
        return lax.bitcast_convert_type(
            a.reshape(n, c // 2, 2), jnp.float32)

    w4, b4 = wcat(layers[0])
    q1, k1, v1, s1 = _tc_qkvs(x, w4, b4)
    kv1 = jnp.concatenate([pk(k1), pk(v1)], axis=1)
    op1, den1 = _sc_conv(q1, kv1, de2, se2, z2, z1)

    w42, b42 = wcat(layers[1])
    q2, k2, v2, s2 = _tc_epi_qkvs(op1, den1, s1, w42, b42, relu=True)
    kv2 = jnp.concatenate([pk(k2), pk(v2)], axis=1)
    op2, den2 = _sc_conv(q2, kv2, de2, se2, z2, z1)

    w1 = lp["W1"]
    w_pq = jnp.concatenate([w1[:c, :], w1[c:, :]], axis=1)
    b_pq = jnp.concatenate([lp["b1"], jnp.zeros((c,), jnp.float32)])
    pmat, qmat = _tc_epi_pq(op2, den2, s2, w_pq, b_pq)
    pq = jnp.concatenate([pk(pmat), pk(qmat)], axis=1)

    lpad = ((l + NW * CB - 1) // (NW * CB)) * (NW * CB)
    pad = lpad - l
    srcp = jnp.concatenate([src, jnp.zeros((pad,), jnp.int32)])
    dstp = jnp.concatenate([dst, jnp.zeros((pad,), jnp.int32)])
    src2 = srcp.reshape(NW, lpad // (NW * CB), CB)
    dst2 = dstp.reshape(NW, lpad // (NW * CB), CB)
    wb = jnp.concatenate(
        [lp["W2"][lperm, 0], jnp.full((LANE,), lp["b2"][0], jnp.float32)])

    out = _sc_link(pq, src2, dst2, wb)
    return out[:l].reshape(l, 1)
